# baseline (speedup 1.0000x reference)
import functools
import jax
import jax.numpy as jnp
from jax import lax
from jax.experimental import pallas as pl
from jax.experimental.pallas import tpu as pltpu
from jax.experimental.pallas import tpu_sc as plsc

D = 64
LANES = 128
CH_VECS = 4
CHUNK = CH_VECS * LANES


def _make_kernel(rows, num_workers):
    rows_per_w = rows // num_workers
    nch = rows_per_w // CHUNK
    iv_per_w = rows_per_w // LANES

    mesh = plsc.VectorSubcoreMesh(core_axis_name="c", subcore_axis_name="s")

    @functools.partial(
        pl.kernel,
        mesh=mesh,
        out_type=jax.ShapeDtypeStruct((rows, D), jnp.float32),
        scratch_types=[
            pltpu.VMEM((CH_VECS, LANES), jnp.int32),
            pltpu.VMEM((CHUNK, D), jnp.float32),
            pltpu.SemaphoreType.DMA,
        ],
        compiler_params=pltpu.CompilerParams(use_tc_tiling_on_sc=False),
    )
    def emb(table_hbm, idx_hbm, out_hbm, idx_v, rows_v, gsem):
        cid = lax.axis_index("c")
        sid = lax.axis_index("s")
        wid = sid * 2 + cid
        base = wid * rows_per_w
        ivbase = wid * iv_per_w

        def body(c, carry):
            r0 = base + c * CHUNK
            iv0 = ivbase + c * CH_VECS
            pltpu.sync_copy(idx_hbm.at[pl.ds(iv0, CH_VECS)], idx_v)
            for j in range(CH_VECS):
                pltpu.async_copy(
                    table_hbm.at[idx_v.at[j]],
                    rows_v.at[pl.ds(j * LANES, LANES)],
                    gsem,
                )
            for j in range(CH_VECS):
                pltpu.make_async_copy(
                    table_hbm.at[idx_v.at[j]],
                    rows_v.at[pl.ds(j * LANES, LANES)],
                    gsem,
                ).wait()
            pltpu.sync_copy(rows_v, out_hbm.at[pl.ds(r0, CHUNK)])
            return carry

        lax.fori_loop(0, nch, body, 0)

    return emb


def kernel(token_ids, weight):
    B, L = token_ids.shape
    rows = B * L
    idx2d = token_ids.reshape(rows // LANES, LANES).astype(jnp.int32)
    emb = _make_kernel(rows, 32)
    out = emb(weight, idx2d)
    return out.reshape(B, L, D)


# speedup vs baseline: 1.0346x; 1.0346x over previous
import functools
import jax
import jax.numpy as jnp
from jax import lax
from jax.experimental import pallas as pl
from jax.experimental.pallas import tpu as pltpu
from jax.experimental.pallas import tpu_sc as plsc

D = 64
LANES = 128
CH_VECS = 4
CHUNK = CH_VECS * LANES
NBUF = 2


def _make_kernel(rows, num_workers):
    rows_per_w = rows // num_workers
    nch = rows_per_w // CHUNK
    iv_per_w = rows_per_w // LANES
    assert nch % NBUF == 0

    mesh = plsc.VectorSubcoreMesh(core_axis_name="c", subcore_axis_name="s")

    @functools.partial(
        pl.kernel,
        mesh=mesh,
        out_type=jax.ShapeDtypeStruct((rows, D), jnp.float32),
        scratch_types=[
            pltpu.VMEM((iv_per_w, LANES), jnp.int32),
            pltpu.VMEM((NBUF, CHUNK, D), jnp.float32),
            pltpu.SemaphoreType.DMA,
            pltpu.SemaphoreType.DMA,
            pltpu.SemaphoreType.DMA,
            pltpu.SemaphoreType.DMA,
        ],
        compiler_params=pltpu.CompilerParams(use_tc_tiling_on_sc=False),
    )
    def emb(table_hbm, idx_hbm, out_hbm, idx_v, rows_v, g0, g1, s0, s1):
        cid = lax.axis_index("c")
        sid = lax.axis_index("s")
        wid = sid * 2 + cid
        base = wid * rows_per_w
        ivbase = wid * iv_per_w
        gsem = (g0, g1)
        ssem = (s0, s1)

        pltpu.sync_copy(idx_hbm.at[pl.ds(ivbase, iv_per_w)], idx_v)

        def fire_g(c, b):
            for j in range(CH_VECS):
                pltpu.async_copy(
                    table_hbm.at[idx_v.at[c * CH_VECS + j]],
                    rows_v.at[b, pl.ds(j * LANES, LANES)],
                    gsem[b],
                )

        def drain_g(c, b):
            for j in range(CH_VECS):
                pltpu.make_async_copy(
                    table_hbm.at[idx_v.at[c * CH_VECS + j]],
                    rows_v.at[b, pl.ds(j * LANES, LANES)],
                    gsem[b],
                ).wait()

        def fire_s(c, b):
            pltpu.async_copy(
                rows_v.at[b], out_hbm.at[pl.ds(base + c * CHUNK, CHUNK)], ssem[b]
            )

        def drain_s(c, b):
            pltpu.make_async_copy(
                rows_v.at[b], out_hbm.at[pl.ds(base + c * CHUNK, CHUNK)], ssem[b]
            ).wait()

        fire_g(0, 0)
        fire_g(1, 1)

        def body(g, carry):
            drain_g(g, 0)
            fire_s(g, 0)
            drain_g(g + 1, 1)
            fire_s(g + 1, 1)
            drain_s(g, 0)

            @pl.when(g + 2 < nch)
            def _():
                fire_g(g + 2, 0)

            drain_s(g + 1, 1)

            @pl.when(g + 3 < nch)
            def _():
                fire_g(g + 3, 1)

            return carry

        lax.fori_loop(0, nch // 2, lambda i, c: body(i * 2, c), 0)

    return emb


def kernel(token_ids, weight):
    B, L = token_ids.shape
    rows = B * L
    idx2d = token_ids.reshape(rows // LANES, LANES).astype(jnp.int32)
    emb = _make_kernel(rows, 32)
    out = emb(weight, idx2d)
    return out.reshape(B, L, D)
